# baseline (speedup 1.0000x reference)
import jax
import jax.numpy as jnp
from jax.experimental import pallas as pl


def _body(ids_ref, w0_ref, w1_ref, out_ref):
    ids3 = ids_ref[...][:, :, None]
    out_ref[...] = jnp.where(ids3 > 0, w1_ref[...], w0_ref[...])


def kernel(id1, W):
    N, J = id1.shape
    D = W.shape[1]
    B = 64
    w0 = W[0][None, None, :]
    w1 = W[1][None, None, :]
    out = pl.pallas_call(
        _body,
        grid=(N // B,),
        in_specs=[
            pl.BlockSpec((B, J), lambda i: (i, 0)),
            pl.BlockSpec((1, 1, D), lambda i: (0, 0, 0)),
            pl.BlockSpec((1, 1, D), lambda i: (0, 0, 0)),
        ],
        out_specs=pl.BlockSpec((B, J, D), lambda i: (i, 0, 0)),
        out_shape=jax.ShapeDtypeStruct((N, J, D), jnp.float32),
    )(id1, w0, w1)
    return out


# speedup vs baseline: 52.7317x; 52.7317x over previous
import jax
import jax.numpy as jnp
from jax.experimental import pallas as pl


def _body(ids_ref, w_ref, out_ref):
    mask = ids_ref[...] > 0
    for d in range(out_ref.shape[0]):
        out_ref[d] = jnp.where(mask, w_ref[1, d], w_ref[0, d])


def kernel(id1, W):
    N, J = id1.shape
    D = W.shape[1]
    ids_t = id1.T
    Bi = 1024
    out_t = pl.pallas_call(
        _body,
        grid=(N // Bi,),
        in_specs=[
            pl.BlockSpec((J, Bi), lambda i: (0, i)),
            pl.BlockSpec((2, D), lambda i: (0, 0)),
        ],
        out_specs=pl.BlockSpec((D, J, Bi), lambda i: (0, 0, i)),
        out_shape=jax.ShapeDtypeStruct((D, J, N), jnp.float32),
    )(ids_t, W)
    return out_t.transpose(2, 1, 0)


# speedup vs baseline: 57.2278x; 1.0853x over previous
import jax
import jax.numpy as jnp
from jax.experimental import pallas as pl


def _body(ids_ref, w_ref, out_ref):
    mask = ids_ref[...] > 0
    for d in range(out_ref.shape[0]):
        out_ref[d] = jnp.where(mask, w_ref[1, d], w_ref[0, d])


def kernel(id1, W):
    N, J = id1.shape
    D = W.shape[1]
    ids_t = id1.T
    Bi = 2048
    out_t = pl.pallas_call(
        _body,
        grid=(N // Bi,),
        in_specs=[
            pl.BlockSpec((J, Bi), lambda i: (0, i)),
            pl.BlockSpec((2, D), lambda i: (0, 0)),
        ],
        out_specs=pl.BlockSpec((D, J, Bi), lambda i: (0, 0, i)),
        out_shape=jax.ShapeDtypeStruct((D, J, N), jnp.float32),
    )(ids_t, W)
    return out_t.transpose(2, 1, 0)
